# baseline (speedup 1.0000x reference)
import functools

import jax
import jax.numpy as jnp
import numpy as np
from jax import lax
from jax.experimental import pallas as pl
from jax.experimental.pallas import tpu as pltpu
from jax.experimental.pallas import tpu_sc as plsc

B = 32
N_EXAMPLES = 50000
N_CLASSES = 64
LANES = 16

ROWS_PER_CHUNK = 400
CHUNK_ELEMS = ROWS_PER_CHUNK * N_CLASSES
N_CHUNKS = N_EXAMPLES // ROWS_PER_CHUNK
ROW_UNROLL = 4


def _body(x_hbm, out_hbm, buf0, buf1, hist, out_v, scr, sem0, sem1):
    wid = lax.axis_index("s") * 2 + lax.axis_index("c")
    iota = lax.broadcasted_iota(jnp.int32, (LANES,), 0)
    zeros = jnp.zeros((LANES,), jnp.float32)
    ones = jnp.ones((LANES,), jnp.float32)
    lane0 = iota == 0
    xor_idx = [iota ^ sh for sh in (8, 4, 2, 1)]

    for j in range(N_CLASSES // LANES):
        hist[pl.ds(j * LANES, LANES)] = zeros

    def start(i, buf, sem):
        pltpu.async_copy(x_hbm.at[wid, i], buf, sem)

    def wait(buf, sem):
        pltpu.make_async_copy(x_hbm.at[wid, 0], buf, sem).wait()

    def one_row(buf, scr, off, j):
        q = [buf[pl.ds(off + k * LANES, LANES)] for k in range(4)]
        m = jnp.maximum(jnp.maximum(q[0], q[1]), jnp.maximum(q[2], q[3]))
        for p in xor_idx:
            scr[pl.ds(j * LANES, LANES)] = m
            t = plsc.load_gather(scr, [p + j * LANES])
            m = jnp.maximum(m, t)
        f = [plsc.all_reduce_ffs(q[k] == m) for k in range(4)]
        h = [jnp.where(f[k] == LANES, N_CLASSES, f[k] + k * LANES)
             for k in range(3)]
        cls = jnp.minimum(jnp.minimum(h[0], h[1]),
                          jnp.minimum(h[2], f[3] + 3 * LANES))
        cls = jnp.broadcast_to(cls, (LANES,)).astype(jnp.int32)
        plsc.addupdate_scatter(hist, [cls], ones, mask=lane0)

    def process(buf):
        def row_body(r, _):
            base = r * (ROW_UNROLL * N_CLASSES)
            for j in range(ROW_UNROLL):
                one_row(buf, scr, base + j * N_CLASSES, j)
            return 0

        lax.fori_loop(0, ROWS_PER_CHUNK // ROW_UNROLL, row_body, 0)

    start(0, buf0, sem0)

    def pair_body(k, _):
        start(2 * k + 1, buf1, sem1)
        wait(buf0, sem0)
        process(buf0)
        start(2 * k + 2, buf0, sem0)
        wait(buf1, sem1)
        process(buf1)
        return 0

    lax.fori_loop(0, (N_CHUNKS - 1) // 2, pair_body, 0)
    wait(buf0, sem0)
    process(buf0)

    scale = jnp.float32(1.0 / N_EXAMPLES)
    for j in range(N_CLASSES // LANES):
        out_v[pl.ds(j * LANES, LANES)] = hist[pl.ds(j * LANES, LANES)] * scale

    pltpu.sync_copy(out_v, out_hbm.at[wid])


def kernel(input):
    x3 = input.reshape(B, N_CHUNKS, CHUNK_ELEMS)
    mesh = plsc.VectorSubcoreMesh(core_axis_name="c", subcore_axis_name="s")
    k = functools.partial(
        pl.kernel,
        out_type=jax.ShapeDtypeStruct((B, N_CLASSES), jnp.float32),
        mesh=mesh,
        scratch_types=[
            pltpu.VMEM((CHUNK_ELEMS,), jnp.float32),
            pltpu.VMEM((CHUNK_ELEMS,), jnp.float32),
            pltpu.VMEM((N_CLASSES,), jnp.float32),
            pltpu.VMEM((N_CLASSES,), jnp.float32),
            pltpu.VMEM((ROW_UNROLL * LANES,), jnp.float32),
            pltpu.SemaphoreType.DMA,
            pltpu.SemaphoreType.DMA,
        ],
        compiler_params=pltpu.CompilerParams(needs_layout_passes=False),
    )(_body)
    return k(x3)


# speedup vs baseline: 1.8766x; 1.8766x over previous
import functools

import jax
import jax.numpy as jnp
import numpy as np
from jax import lax
from jax.experimental import pallas as pl
from jax.experimental.pallas import tpu as pltpu
from jax.experimental.pallas import tpu_sc as plsc

B = 32
N_EXAMPLES = 50000
N_CLASSES = 64
LANES = 16

ROWS_PER_CHUNK = 400
CHUNK_ELEMS = ROWS_PER_CHUNK * N_CLASSES
N_CHUNKS = N_EXAMPLES // ROWS_PER_CHUNK
ROW_UNROLL = 4


def _body(x_hbm, out_hbm, buf0, buf1, hist, out_v, scr, sem0, sem1):
    wid = lax.axis_index("s") * 2 + lax.axis_index("c")
    iota = lax.broadcasted_iota(jnp.int32, (LANES,), 0)
    zeros = jnp.zeros((LANES,), jnp.float32)
    ones = jnp.ones((LANES,), jnp.float32)
    lane0 = iota == 0
    xor_idx = [iota ^ sh for sh in (8, 4, 2, 1)]

    for j in range(N_CLASSES // LANES):
        hist[pl.ds(j * LANES, LANES)] = zeros

    def start(i, buf, sem):
        pltpu.async_copy(x_hbm.at[wid, i], buf, sem)

    def wait(buf, sem):
        pltpu.make_async_copy(x_hbm.at[wid, 0], buf, sem).wait()

    def one_row(buf, r):
        off = r * N_CLASSES
        sbase = r * LANES
        q = [buf[pl.ds(off + k * LANES, LANES)] for k in range(4)]
        m = jnp.maximum(jnp.maximum(q[0], q[1]), jnp.maximum(q[2], q[3]))
        for p in xor_idx:
            scr[pl.ds(sbase, LANES)] = m
            t = plsc.load_gather(scr, [p + sbase])
            m = jnp.maximum(m, t)
        f = [plsc.all_reduce_ffs(q[k] == m) for k in range(4)]
        h = [jnp.where(f[k] == LANES, N_CLASSES, f[k] + k * LANES)
             for k in range(3)]
        cls = jnp.minimum(jnp.minimum(h[0], h[1]),
                          jnp.minimum(h[2], f[3] + 3 * LANES))
        cls = jnp.broadcast_to(cls, (LANES,)).astype(jnp.int32)
        plsc.addupdate_scatter(hist, [cls], ones, mask=lane0)

    def process(buf):
        @plsc.parallel_loop(0, ROWS_PER_CHUNK, unroll=ROW_UNROLL)
        def _rows(r):
            one_row(buf, r)

    start(0, buf0, sem0)

    def pair_body(k, _):
        start(2 * k + 1, buf1, sem1)
        wait(buf0, sem0)
        process(buf0)
        start(2 * k + 2, buf0, sem0)
        wait(buf1, sem1)
        process(buf1)
        return 0

    lax.fori_loop(0, (N_CHUNKS - 1) // 2, pair_body, 0)
    wait(buf0, sem0)
    process(buf0)

    scale = jnp.float32(1.0 / N_EXAMPLES)
    for j in range(N_CLASSES // LANES):
        out_v[pl.ds(j * LANES, LANES)] = hist[pl.ds(j * LANES, LANES)] * scale

    pltpu.sync_copy(out_v, out_hbm.at[wid])


def kernel(input):
    x3 = input.reshape(B, N_CHUNKS, CHUNK_ELEMS)
    mesh = plsc.VectorSubcoreMesh(core_axis_name="c", subcore_axis_name="s")
    k = functools.partial(
        pl.kernel,
        out_type=jax.ShapeDtypeStruct((B, N_CLASSES), jnp.float32),
        mesh=mesh,
        scratch_types=[
            pltpu.VMEM((CHUNK_ELEMS,), jnp.float32),
            pltpu.VMEM((CHUNK_ELEMS,), jnp.float32),
            pltpu.VMEM((N_CLASSES,), jnp.float32),
            pltpu.VMEM((N_CLASSES,), jnp.float32),
            pltpu.VMEM((ROWS_PER_CHUNK * LANES,), jnp.float32),
            pltpu.SemaphoreType.DMA,
            pltpu.SemaphoreType.DMA,
        ],
        compiler_params=pltpu.CompilerParams(needs_layout_passes=False),
    )(_body)
    return k(x3)
